# Initial kernel scaffold; baseline (speedup 1.0000x reference)
import functools

import jax
import jax.numpy as jnp
from jax import lax
from jax.experimental import pallas as pl
from jax.experimental.pallas import tpu as pltpu
from jax.experimental.pallas import tpu_sc as plsc

NC = 2
NS = 16
NW = NC * NS
L = 16
C = 128

_MESH = plsc.VectorSubcoreMesh(core_axis_name="c", subcore_axis_name="s")


def _bcast_lane(vec, e):
  idx = jnp.full((L,), e, dtype=jnp.int32)
  return jnp.take_along_axis(vec, idx, axis=0, mode="promise_in_bounds")


def _make_deg_kernel(G, Np):
  @functools.partial(
      pl.kernel,
      mesh=_MESH,
      out_type=jax.ShapeDtypeStruct((NC, Np), jnp.float32),
      scratch_types=[
          pltpu.VMEM((G, C), jnp.int32),
          pltpu.VMEM((G, C), jnp.float32),
          pltpu.VMEM((640,), jnp.float32),
          pltpu.VMEM_SHARED((Np,), jnp.float32),
          pltpu.SemaphoreType.DMA,
          pltpu.SemaphoreType.DMA,
      ],
  )
  def deg_kernel(dst_hbm, ew_hbm, out_hbm, dst_v, ew_v, zbuf, acc, sem0, sem1):
    c = lax.axis_index("c")
    s = lax.axis_index("s")
    wid = s * NC + c
    stripe = Np // NS
    r0 = s * stripe

    pltpu.sync_copy(dst_hbm.at[wid], dst_v)
    pltpu.sync_copy(ew_hbm.at[wid], ew_v)

    z = jnp.zeros((L,), jnp.float32)

    def zb(i, carry):
      zbuf[pl.ds(i * L, L)] = z
      return carry

    lax.fori_loop(0, 640 // L, zb, 0)
    for k in range(stripe // 640):
      pltpu.sync_copy(zbuf, acc.at[pl.ds(r0 + k * 640, 640)])
    plsc.subcore_barrier()

    pltpu.async_copy(ew_v.at[0], acc.at[dst_v.at[0]], sem0, add=True)

    def body(i, carry):
      a = 2 * i
      b = a + 1
      pltpu.async_copy(ew_v.at[b], acc.at[dst_v.at[b]], sem1, add=True)
      pltpu.make_async_copy(ew_v.at[a], acc.at[dst_v.at[a]], sem0).wait()

      @pl.when(i < G // 2 - 1)
      def _():
        pltpu.async_copy(ew_v.at[a + 2], acc.at[dst_v.at[a + 2]], sem0,
                         add=True)

      pltpu.make_async_copy(ew_v.at[b], acc.at[dst_v.at[b]], sem1).wait()
      return carry

    lax.fori_loop(0, G // 2, body, 0)
    plsc.subcore_barrier()
    pltpu.sync_copy(acc.at[pl.ds(r0, stripe)], out_hbm.at[c, pl.ds(r0, stripe)])

  return deg_kernel


def _make_prop_kernel(G, Np):
  @functools.partial(
      pl.kernel,
      mesh=_MESH,
      out_type=jax.ShapeDtypeStruct((NC, Np, 128), jnp.float32),
      scratch_types=[
          pltpu.VMEM((G, C), jnp.int32),
          pltpu.VMEM((G, C), jnp.int32),
          pltpu.VMEM((G, C), jnp.float32),
          pltpu.VMEM((C, 128), jnp.float32),
          pltpu.VMEM((C, 128), jnp.float32),
          pltpu.VMEM_SHARED((Np, 128), jnp.float32),
          pltpu.SemaphoreType.DMA,
          pltpu.SemaphoreType.DMA,
      ],
  )
  def prop_kernel(y_hbm, src_hbm, dst_hbm, ew_hbm, out_hbm,
                  src_v, dst_v, ew_v, buf0, buf1, acc, sem0, sem1):
    c = lax.axis_index("c")
    s = lax.axis_index("s")
    wid = s * NC + c
    stripe = Np // NS
    r0 = s * stripe

    pltpu.sync_copy(src_hbm.at[wid], src_v)
    pltpu.sync_copy(dst_hbm.at[wid], dst_v)
    pltpu.sync_copy(ew_hbm.at[wid], ew_v)

    z = jnp.zeros((L,), jnp.float32)

    def zrow(r, carry):
      for f in range(128 // L):
        buf0[r, pl.ds(f * L, L)] = z
      return carry

    lax.fori_loop(0, C, zrow, 0)
    for k in range(stripe // C):
      pltpu.sync_copy(buf0, acc.at[pl.ds(r0 + k * C, C)])
    plsc.subcore_barrier()

    def scale(buf, a):
      def group(j, carry):
        ew_vec = ew_v[a, pl.ds(j * L, L)]
        for e in range(L):
          bv = _bcast_lane(ew_vec, e)
          r = j * L + e
          for f in range(128 // L):
            buf[r, pl.ds(f * L, L)] = buf[r, pl.ds(f * L, L)] * bv
        return carry

      lax.fori_loop(0, C // L, group, 0)

    pltpu.async_copy(y_hbm.at[src_v.at[0]], buf0, sem0)

    def body(i, carry):
      a = 2 * i
      b = a + 1
      pltpu.async_copy(y_hbm.at[src_v.at[b]], buf1, sem1)
      pltpu.make_async_copy(y_hbm.at[src_v.at[a]], buf0, sem0).wait()
      scale(buf0, a)
      pltpu.sync_copy(buf0, acc.at[dst_v.at[a]], add=True)

      @pl.when(i < G // 2 - 1)
      def _():
        pltpu.async_copy(y_hbm.at[src_v.at[a + 2]], buf0, sem0)

      pltpu.make_async_copy(y_hbm.at[src_v.at[b]], buf1, sem1).wait()
      scale(buf1, b)
      pltpu.sync_copy(buf1, acc.at[dst_v.at[b]], add=True)
      return carry

    lax.fori_loop(0, G // 2, body, 0)
    plsc.subcore_barrier()
    for k in range(stripe // C):
      pltpu.sync_copy(acc.at[pl.ds(r0 + k * C, C)],
                      out_hbm.at[c, pl.ds(r0 + k * C, C)])

  return prop_kernel


def _dis(d0_blk, d1_blk):
  return lax.rsqrt(d0_blk + d1_blk + 1.0)


def _tc_scale_body(x_blk, d0_blk, d1_blk, y_blk):
  y_blk[...] = x_blk[...] * _dis(d0_blk[...], d1_blk[...])


def _tc_layer1_body(sa, sb, y1, d0, d1, w, b, y2):
  dis = _dis(d0[...], d1[...])
  p = (sa[...] + sb[...] + y1[...]) * dis
  h = jnp.dot(p, w[...], preferred_element_type=jnp.float32) + b[...]
  y2[...] = jnp.maximum(h, 0.0) * dis


def _tc_layer23_body(sa, sb, y2, d0, d1, wmu, bmu, wlv, blv, mu, lv):
  dis = _dis(d0[...], d1[...])
  p = (sa[...] + sb[...] + y2[...]) * dis
  mu[...] = jnp.dot(p, wmu[...], preferred_element_type=jnp.float32) + bmu[...]
  lv[...] = jnp.dot(p, wlv[...], preferred_element_type=jnp.float32) + blv[...]


def _row_spec(rb, cols):
  return pl.BlockSpec((rb, cols), lambda i: (i, 0))


def _full_spec(shape):
  return pl.BlockSpec(shape, lambda i: tuple(0 for _ in shape))


def kernel(x, edge_index, edge_weight, W1, b1, Wmu, bmu, Wlv, blv):
  N, D = x.shape
  E = edge_index.shape[1]
  Z = Wmu.shape[1]

  per = NW * C
  G = -(-E // per)
  if G % 2:
    G += 1
  Ep = G * per
  Np = -(-N // (NS * 128)) * (NS * 128)

  src = edge_index[0]
  dst = edge_index[1]
  pad = Ep - E
  src_r = jnp.concatenate([src, jnp.zeros((pad,), src.dtype)]).reshape(NW, G, C)
  dst_r = jnp.concatenate([dst, jnp.zeros((pad,), dst.dtype)]).reshape(NW, G, C)
  ew_r = jnp.concatenate(
      [edge_weight, jnp.zeros((pad,), edge_weight.dtype)]).reshape(NW, G, C)
  src_r = src_r.astype(jnp.int32)
  dst_r = dst_r.astype(jnp.int32)

  deg_p = _make_deg_kernel(G, Np)(dst_r, ew_r)
  d0 = deg_p[0][:, None]
  d1 = deg_p[1][:, None]

  RB = 500
  grid = (N // RB,)

  y1 = pl.pallas_call(
      _tc_scale_body,
      grid=grid,
      in_specs=[_row_spec(RB, D), _row_spec(RB, 1), _row_spec(RB, 1)],
      out_specs=_row_spec(RB, D),
      out_shape=jax.ShapeDtypeStruct((Np, D), jnp.float32),
  )(x, d0, d1)

  prop = _make_prop_kernel(G, Np)
  S1 = prop(y1, src_r, dst_r, ew_r)

  y2 = pl.pallas_call(
      _tc_layer1_body,
      grid=grid,
      in_specs=[
          _row_spec(RB, D), _row_spec(RB, D), _row_spec(RB, D),
          _row_spec(RB, 1), _row_spec(RB, 1),
          _full_spec((D, D)), _full_spec((1, D)),
      ],
      out_specs=_row_spec(RB, D),
      out_shape=jax.ShapeDtypeStruct((Np, D), jnp.float32),
  )(S1[0], S1[1], y1, d0, d1, W1, b1[None, :])

  S2 = prop(y2, src_r, dst_r, ew_r)

  mu, lv = pl.pallas_call(
      _tc_layer23_body,
      grid=grid,
      in_specs=[
          _row_spec(RB, D), _row_spec(RB, D), _row_spec(RB, D),
          _row_spec(RB, 1), _row_spec(RB, 1),
          _full_spec((D, Z)), _full_spec((1, Z)),
          _full_spec((D, Z)), _full_spec((1, Z)),
      ],
      out_specs=[_row_spec(RB, Z), _row_spec(RB, Z)],
      out_shape=[
          jax.ShapeDtypeStruct((N, Z), jnp.float32),
          jax.ShapeDtypeStruct((N, Z), jnp.float32),
      ],
  )(S2[0], S2[1], y2, d0, d1, Wmu, bmu[None, :], Wlv, blv[None, :])

  return (mu, lv)


# speedup vs baseline: 13.8968x; 13.8968x over previous
import functools

import jax
import jax.numpy as jnp
from jax import lax
from jax.experimental import pallas as pl
from jax.experimental.pallas import tpu as pltpu
from jax.experimental.pallas import tpu_sc as plsc

NC = 2
NS = 16
NW = NC * NS
L = 16
C = 128

_MESH = plsc.VectorSubcoreMesh(core_axis_name="c", subcore_axis_name="s")


def _bcast_lane(vec, e):
  idx = jnp.full((L,), e, dtype=jnp.int32)
  return jnp.take_along_axis(vec, idx, axis=0, mode="promise_in_bounds")


def _make_deg_kernel(G, Np):
  @functools.partial(
      pl.kernel,
      mesh=_MESH,
      out_type=jax.ShapeDtypeStruct((NC, Np), jnp.float32),
      scratch_types=[
          pltpu.VMEM((G, C), jnp.int32),
          pltpu.VMEM((G, C), jnp.float32),
          pltpu.VMEM((640,), jnp.float32),
          pltpu.VMEM_SHARED((Np,), jnp.float32),
          pltpu.SemaphoreType.DMA,
          pltpu.SemaphoreType.DMA,
      ],
  )
  def deg_kernel(dst_hbm, ew_hbm, out_hbm, dst_v, ew_v, zbuf, acc, sem0, sem1):
    c = lax.axis_index("c")
    s = lax.axis_index("s")
    wid = s * NC + c
    stripe = Np // NS
    r0 = s * stripe

    pltpu.sync_copy(dst_hbm.at[wid], dst_v)
    pltpu.sync_copy(ew_hbm.at[wid], ew_v)

    z = jnp.zeros((L,), jnp.float32)

    def zb(i, carry):
      zbuf[pl.ds(i * L, L)] = z
      return carry

    lax.fori_loop(0, 640 // L, zb, 0)
    for k in range(stripe // 640):
      pltpu.sync_copy(zbuf, acc.at[pl.ds(r0 + k * 640, 640)])
    plsc.subcore_barrier()

    pltpu.async_copy(ew_v.at[0], acc.at[dst_v.at[0]], sem0, add=True)

    def body(i, carry):
      a = 2 * i
      b = a + 1
      pltpu.async_copy(ew_v.at[b], acc.at[dst_v.at[b]], sem1, add=True)
      pltpu.make_async_copy(ew_v.at[a], acc.at[dst_v.at[a]], sem0).wait()

      @pl.when(i < G // 2 - 1)
      def _():
        pltpu.async_copy(ew_v.at[a + 2], acc.at[dst_v.at[a + 2]], sem0,
                         add=True)

      pltpu.make_async_copy(ew_v.at[b], acc.at[dst_v.at[b]], sem1).wait()
      return carry

    lax.fori_loop(0, G // 2, body, 0)
    plsc.subcore_barrier()
    pltpu.sync_copy(acc.at[pl.ds(r0, stripe)], out_hbm.at[c, pl.ds(r0, stripe)])

  return deg_kernel


def _make_prop_kernel(G, Np):
  @functools.partial(
      pl.kernel,
      mesh=_MESH,
      out_type=jax.ShapeDtypeStruct((Np, 128), jnp.float32),
      scratch_types=[
          pltpu.VMEM((2, C), jnp.int32),
          pltpu.VMEM((2, C), jnp.int32),
          pltpu.VMEM((1, C), jnp.float32),
          pltpu.VMEM((1, C), jnp.float32),
          pltpu.VMEM((C, 128), jnp.float32),
          pltpu.VMEM((C, 128), jnp.float32),
          pltpu.VMEM_SHARED((Np, 128), jnp.float32),
          pltpu.SemaphoreType.DMA,
          pltpu.SemaphoreType.DMA,
          pltpu.SemaphoreType.DMA,
          pltpu.SemaphoreType.DMA,
          pltpu.SemaphoreType.DMA,
          pltpu.SemaphoreType.DMA,
      ],
  )
  def prop_kernel(y_hbm, pk_hbm, ew_hbm, out_hbm,
                  pk0, pk1, ew0, ew1, buf0, buf1, acc,
                  psem0, psem1, gsem0, gsem1, ssem0, ssem1):
    c = lax.axis_index("c")
    s = lax.axis_index("s")

    @pl.when(c == 0)
    def _sc0_work():
      wid = s
      stripe = Np // NS
      r0 = s * stripe

      z = jnp.zeros((L,), jnp.float32)

      def zrow(r, carry):
        for f in range(128 // L):
          buf0[r, pl.ds(f * L, L)] = z
        return carry

      lax.fori_loop(0, C, zrow, 0)
      for k in range(stripe // C):
        pltpu.sync_copy(buf0, acc.at[pl.ds(r0 + k * C, C)])
      plsc.subcore_barrier()

      def scale(buf, ew):
        def group(j, carry):
          ew_vec = ew[0, pl.ds(j * L, L)]
          for e in range(L):
            bv = _bcast_lane(ew_vec, e)
            r = j * L + e
            for f in range(128 // L):
              buf[r, pl.ds(f * L, L)] = buf[r, pl.ds(f * L, L)] * bv
          return carry

        lax.fori_loop(0, C // L, group, 0)

      pltpu.sync_copy(pk_hbm.at[wid, 0], pk0)
      pltpu.sync_copy(ew_hbm.at[wid, 0], ew0)
      pltpu.async_copy(pk_hbm.at[wid, 1], pk1, psem1)
      pltpu.async_copy(ew_hbm.at[wid, 1], ew1, psem1)
      pltpu.async_copy(y_hbm.at[pk0.at[0]], buf0, gsem0)

      def body(i, carry):
        a = 2 * i
        b = a + 1
        pltpu.make_async_copy(pk_hbm.at[wid, b], pk1, psem1).wait()
        pltpu.make_async_copy(ew_hbm.at[wid, b], ew1, psem1).wait()

        @pl.when(i > 0)
        def _():
          pltpu.make_async_copy(buf1, acc.at[pk1.at[1]], ssem1).wait()

        pltpu.async_copy(y_hbm.at[pk1.at[0]], buf1, gsem1)
        pltpu.make_async_copy(y_hbm.at[pk0.at[0]], buf0, gsem0).wait()
        scale(buf0, ew0)
        pltpu.async_copy(buf0, acc.at[pk0.at[1]], ssem0, add=True)

        @pl.when(a + 2 < G)
        def _():
          pltpu.async_copy(pk_hbm.at[wid, a + 2], pk0, psem0)
          pltpu.async_copy(ew_hbm.at[wid, a + 2], ew0, psem0)

        pltpu.make_async_copy(y_hbm.at[pk1.at[0]], buf1, gsem1).wait()
        scale(buf1, ew1)
        pltpu.async_copy(buf1, acc.at[pk1.at[1]], ssem1, add=True)

        @pl.when(a + 2 < G)
        def _():
          pltpu.make_async_copy(pk_hbm.at[wid, a + 2], pk0, psem0).wait()
          pltpu.make_async_copy(ew_hbm.at[wid, a + 2], ew0, psem0).wait()
          pltpu.make_async_copy(buf0, acc.at[pk0.at[1]], ssem0).wait()
          pltpu.async_copy(y_hbm.at[pk0.at[0]], buf0, gsem0)

        @pl.when(b + 2 < G)
        def _():
          pltpu.async_copy(pk_hbm.at[wid, b + 2], pk1, psem1)
          pltpu.async_copy(ew_hbm.at[wid, b + 2], ew1, psem1)

        return carry

      lax.fori_loop(0, G // 2, body, 0)
      pltpu.make_async_copy(buf0, acc.at[pk0.at[1]], ssem0).wait()
      pltpu.make_async_copy(buf1, acc.at[pk1.at[1]], ssem1).wait()
      plsc.subcore_barrier()
      for k in range(stripe // C):
        pltpu.sync_copy(acc.at[pl.ds(r0 + k * C, C)],
                        out_hbm.at[pl.ds(r0 + k * C, C)])

  return prop_kernel


def _dis(d0_blk, d1_blk):
  return lax.rsqrt(d0_blk + d1_blk + 1.0)


def _tc_scale_body(x_blk, d0_blk, d1_blk, y_blk):
  y_blk[...] = x_blk[...] * _dis(d0_blk[...], d1_blk[...])


def _tc_layer1_body(sa, y1, d0, d1, w, b, y2):
  dis = _dis(d0[...], d1[...])
  p = (sa[...] + y1[...]) * dis
  h = jnp.dot(p, w[...], preferred_element_type=jnp.float32) + b[...]
  y2[...] = jnp.maximum(h, 0.0) * dis


def _tc_layer23_body(sa, y2, d0, d1, wmu, bmu, wlv, blv, mu, lv):
  dis = _dis(d0[...], d1[...])
  p = (sa[...] + y2[...]) * dis
  mu[...] = jnp.dot(p, wmu[...], preferred_element_type=jnp.float32) + bmu[...]
  lv[...] = jnp.dot(p, wlv[...], preferred_element_type=jnp.float32) + blv[...]


def _row_spec(rb, cols):
  return pl.BlockSpec((rb, cols), lambda i: (i, 0))


def _full_spec(shape):
  return pl.BlockSpec(shape, lambda i: tuple(0 for _ in shape))


def _pad_reshape(a, ep, shape):
  pad = ep - a.shape[0]
  return jnp.concatenate([a, jnp.zeros((pad,), a.dtype)]).reshape(shape)


def kernel(x, edge_index, edge_weight, W1, b1, Wmu, bmu, Wlv, blv):
  N, D = x.shape
  E = edge_index.shape[1]
  Z = Wmu.shape[1]

  Np = -(-N // (NS * 128)) * (NS * 128)

  src = edge_index[0].astype(jnp.int32)
  dst = edge_index[1].astype(jnp.int32)

  Gd = -(-E // (NW * C))
  if Gd % 2:
    Gd += 1
  dst_d = _pad_reshape(dst, Gd * NW * C, (NW, Gd, C))
  ew_d = _pad_reshape(edge_weight, Gd * NW * C, (NW, Gd, C))

  Gp = -(-E // (NS * C))
  if Gp % 2:
    Gp += 1
  Ep = Gp * NS * C
  src_p = _pad_reshape(src, Ep, (NS, Gp, C))
  dst_p = _pad_reshape(dst, Ep, (NS, Gp, C))
  pk = jnp.stack([src_p, dst_p], axis=2)
  ew_p = _pad_reshape(edge_weight, Ep, (NS, Gp, 1, C))

  deg_p = _make_deg_kernel(Gd, Np)(dst_d, ew_d)
  d0 = deg_p[0][:, None]
  d1 = deg_p[1][:, None]

  RB = 1000
  grid = (N // RB,)

  y1 = pl.pallas_call(
      _tc_scale_body,
      grid=grid,
      in_specs=[_row_spec(RB, D), _row_spec(RB, 1), _row_spec(RB, 1)],
      out_specs=_row_spec(RB, D),
      out_shape=jax.ShapeDtypeStruct((Np, D), jnp.float32),
  )(x, d0, d1)

  prop = _make_prop_kernel(Gp, Np)
  S1 = prop(y1, pk, ew_p)

  y2 = pl.pallas_call(
      _tc_layer1_body,
      grid=grid,
      in_specs=[
          _row_spec(RB, D), _row_spec(RB, D),
          _row_spec(RB, 1), _row_spec(RB, 1),
          _full_spec((D, D)), _full_spec((1, D)),
      ],
      out_specs=_row_spec(RB, D),
      out_shape=jax.ShapeDtypeStruct((Np, D), jnp.float32),
  )(S1, y1, d0, d1, W1, b1[None, :])

  S2 = prop(y2, pk, ew_p)

  mu, lv = pl.pallas_call(
      _tc_layer23_body,
      grid=grid,
      in_specs=[
          _row_spec(RB, D), _row_spec(RB, D),
          _row_spec(RB, 1), _row_spec(RB, 1),
          _full_spec((D, Z)), _full_spec((1, Z)),
          _full_spec((D, Z)), _full_spec((1, Z)),
      ],
      out_specs=[_row_spec(RB, Z), _row_spec(RB, Z)],
      out_shape=[
          jax.ShapeDtypeStruct((N, Z), jnp.float32),
          jax.ShapeDtypeStruct((N, Z), jnp.float32),
      ],
  )(S2, y2, d0, d1, Wmu, bmu[None, :], Wlv, blv[None, :])

  return (mu, lv)


# speedup vs baseline: 13.9002x; 1.0002x over previous
import functools

import jax
import jax.numpy as jnp
from jax import lax
from jax.experimental import pallas as pl
from jax.experimental.pallas import tpu as pltpu
from jax.experimental.pallas import tpu_sc as plsc

NC = 2
NS = 16
NW = NC * NS
L = 16
C = 128

_MESH = plsc.VectorSubcoreMesh(core_axis_name="c", subcore_axis_name="s")


def _bcast_lane(vec, e):
  idx = jnp.full((L,), e, dtype=jnp.int32)
  return jnp.take_along_axis(vec, idx, axis=0, mode="promise_in_bounds")


def _make_deg_kernel(G, Np):
  @functools.partial(
      pl.kernel,
      mesh=_MESH,
      out_type=jax.ShapeDtypeStruct((NC, Np), jnp.float32),
      scratch_types=[
          pltpu.VMEM((G, C), jnp.int32),
          pltpu.VMEM((G, C), jnp.float32),
          pltpu.VMEM((640,), jnp.float32),
          pltpu.VMEM_SHARED((Np,), jnp.float32),
          pltpu.SemaphoreType.DMA,
          pltpu.SemaphoreType.DMA,
      ],
  )
  def deg_kernel(dst_hbm, ew_hbm, out_hbm, dst_v, ew_v, zbuf, acc, sem0, sem1):
    c = lax.axis_index("c")
    s = lax.axis_index("s")
    wid = s * NC + c
    stripe = Np // NS
    r0 = s * stripe

    pltpu.sync_copy(dst_hbm.at[wid], dst_v)
    pltpu.sync_copy(ew_hbm.at[wid], ew_v)

    z = jnp.zeros((L,), jnp.float32)

    def zb(i, carry):
      zbuf[pl.ds(i * L, L)] = z
      return carry

    lax.fori_loop(0, 640 // L, zb, 0)
    for k in range(stripe // 640):
      pltpu.sync_copy(zbuf, acc.at[pl.ds(r0 + k * 640, 640)])
    plsc.subcore_barrier()

    pltpu.async_copy(ew_v.at[0], acc.at[dst_v.at[0]], sem0, add=True)

    def body(i, carry):
      a = 2 * i
      b = a + 1
      pltpu.async_copy(ew_v.at[b], acc.at[dst_v.at[b]], sem1, add=True)
      pltpu.make_async_copy(ew_v.at[a], acc.at[dst_v.at[a]], sem0).wait()

      @pl.when(i < G // 2 - 1)
      def _():
        pltpu.async_copy(ew_v.at[a + 2], acc.at[dst_v.at[a + 2]], sem0,
                         add=True)

      pltpu.make_async_copy(ew_v.at[b], acc.at[dst_v.at[b]], sem1).wait()
      return carry

    lax.fori_loop(0, G // 2, body, 0)
    plsc.subcore_barrier()
    pltpu.sync_copy(acc.at[pl.ds(r0, stripe)], out_hbm.at[c, pl.ds(r0, stripe)])

  return deg_kernel


def _make_prop_kernel(G, Np):
  @functools.partial(
      pl.kernel,
      mesh=_MESH,
      out_type=jax.ShapeDtypeStruct((Np, 128), jnp.float32),
      scratch_types=[
          pltpu.VMEM((2, C), jnp.int32),
          pltpu.VMEM((2, C), jnp.int32),
          pltpu.VMEM((1, C), jnp.float32),
          pltpu.VMEM((1, C), jnp.float32),
          pltpu.VMEM((C, 128), jnp.float32),
          pltpu.VMEM((C, 128), jnp.float32),
          pltpu.VMEM_SHARED((Np, 128), jnp.float32),
          pltpu.SemaphoreType.DMA,
          pltpu.SemaphoreType.DMA,
          pltpu.SemaphoreType.DMA,
          pltpu.SemaphoreType.DMA,
          pltpu.SemaphoreType.DMA,
          pltpu.SemaphoreType.DMA,
      ],
  )
  def prop_kernel(y_hbm, pk_hbm, ew_hbm, out_hbm,
                  pk0, pk1, ew0, ew1, buf0, buf1, acc,
                  psem0, psem1, gsem0, gsem1, ssem0, ssem1):
    c = lax.axis_index("c")
    s = lax.axis_index("s")

    @pl.when(c == 0)
    def _sc0_work():
      wid = s
      stripe = Np // NS
      r0 = s * stripe

      z = jnp.zeros((L,), jnp.float32)

      def zrow(r, carry):
        for f in range(128 // L):
          buf0[r, pl.ds(f * L, L)] = z
        return carry

      lax.fori_loop(0, C, zrow, 0)
      for k in range(stripe // C):
        pltpu.sync_copy(buf0, acc.at[pl.ds(r0 + k * C, C)])
      plsc.subcore_barrier()

      def scale(buf, ew):
        @plsc.parallel_loop(0, C // L, 1, unroll=2)
        def _group(j):
          ew_vec = ew[0, pl.ds(j * L, L)]
          for e in range(L):
            bv = _bcast_lane(ew_vec, e)
            r = j * L + e
            for f in range(128 // L):
              buf[r, pl.ds(f * L, L)] = buf[r, pl.ds(f * L, L)] * bv

      pltpu.sync_copy(pk_hbm.at[wid, 0], pk0)
      pltpu.sync_copy(ew_hbm.at[wid, 0], ew0)
      pltpu.async_copy(pk_hbm.at[wid, 1], pk1, psem1)
      pltpu.async_copy(ew_hbm.at[wid, 1], ew1, psem1)
      pltpu.async_copy(y_hbm.at[pk0.at[0]], buf0, gsem0)

      def body(i, carry):
        a = 2 * i
        b = a + 1
        pltpu.make_async_copy(pk_hbm.at[wid, b], pk1, psem1).wait()
        pltpu.make_async_copy(ew_hbm.at[wid, b], ew1, psem1).wait()

        @pl.when(i > 0)
        def _():
          pltpu.make_async_copy(buf1, acc.at[pk1.at[1]], ssem1).wait()

        pltpu.async_copy(y_hbm.at[pk1.at[0]], buf1, gsem1)
        pltpu.make_async_copy(y_hbm.at[pk0.at[0]], buf0, gsem0).wait()
        scale(buf0, ew0)
        pltpu.async_copy(buf0, acc.at[pk0.at[1]], ssem0, add=True)

        @pl.when(a + 2 < G)
        def _():
          pltpu.async_copy(pk_hbm.at[wid, a + 2], pk0, psem0)
          pltpu.async_copy(ew_hbm.at[wid, a + 2], ew0, psem0)

        pltpu.make_async_copy(y_hbm.at[pk1.at[0]], buf1, gsem1).wait()
        scale(buf1, ew1)
        pltpu.async_copy(buf1, acc.at[pk1.at[1]], ssem1, add=True)

        @pl.when(a + 2 < G)
        def _():
          pltpu.make_async_copy(pk_hbm.at[wid, a + 2], pk0, psem0).wait()
          pltpu.make_async_copy(ew_hbm.at[wid, a + 2], ew0, psem0).wait()
          pltpu.make_async_copy(buf0, acc.at[pk0.at[1]], ssem0).wait()
          pltpu.async_copy(y_hbm.at[pk0.at[0]], buf0, gsem0)

        @pl.when(b + 2 < G)
        def _():
          pltpu.async_copy(pk_hbm.at[wid, b + 2], pk1, psem1)
          pltpu.async_copy(ew_hbm.at[wid, b + 2], ew1, psem1)

        return carry

      lax.fori_loop(0, G // 2, body, 0)
      pltpu.make_async_copy(buf0, acc.at[pk0.at[1]], ssem0).wait()
      pltpu.make_async_copy(buf1, acc.at[pk1.at[1]], ssem1).wait()
      plsc.subcore_barrier()
      for k in range(stripe // C):
        pltpu.sync_copy(acc.at[pl.ds(r0 + k * C, C)],
                        out_hbm.at[pl.ds(r0 + k * C, C)])

  return prop_kernel


def _dis(d0_blk, d1_blk):
  return lax.rsqrt(d0_blk + d1_blk + 1.0)


def _tc_scale_body(x_blk, d0_blk, d1_blk, y_blk):
  y_blk[...] = x_blk[...] * _dis(d0_blk[...], d1_blk[...])


def _tc_layer1_body(sa, y1, d0, d1, w, b, y2):
  dis = _dis(d0[...], d1[...])
  p = (sa[...] + y1[...]) * dis
  h = jnp.dot(p, w[...], preferred_element_type=jnp.float32) + b[...]
  y2[...] = jnp.maximum(h, 0.0) * dis


def _tc_layer23_body(sa, y2, d0, d1, wmu, bmu, wlv, blv, mu, lv):
  dis = _dis(d0[...], d1[...])
  p = (sa[...] + y2[...]) * dis
  mu[...] = jnp.dot(p, wmu[...], preferred_element_type=jnp.float32) + bmu[...]
  lv[...] = jnp.dot(p, wlv[...], preferred_element_type=jnp.float32) + blv[...]


def _row_spec(rb, cols):
  return pl.BlockSpec((rb, cols), lambda i: (i, 0))


def _full_spec(shape):
  return pl.BlockSpec(shape, lambda i: tuple(0 for _ in shape))


def _pad_reshape(a, ep, shape):
  pad = ep - a.shape[0]
  return jnp.concatenate([a, jnp.zeros((pad,), a.dtype)]).reshape(shape)


def kernel(x, edge_index, edge_weight, W1, b1, Wmu, bmu, Wlv, blv):
  N, D = x.shape
  E = edge_index.shape[1]
  Z = Wmu.shape[1]

  Np = -(-N // (NS * 128)) * (NS * 128)

  src = edge_index[0].astype(jnp.int32)
  dst = edge_index[1].astype(jnp.int32)

  Gd = -(-E // (NW * C))
  if Gd % 2:
    Gd += 1
  dst_d = _pad_reshape(dst, Gd * NW * C, (NW, Gd, C))
  ew_d = _pad_reshape(edge_weight, Gd * NW * C, (NW, Gd, C))

  Gp = -(-E // (NS * C))
  if Gp % 2:
    Gp += 1
  Ep = Gp * NS * C
  src_p = _pad_reshape(src, Ep, (NS, Gp, C))
  dst_p = _pad_reshape(dst, Ep, (NS, Gp, C))
  pk = jnp.stack([src_p, dst_p], axis=2)
  ew_p = _pad_reshape(edge_weight, Ep, (NS, Gp, 1, C))

  deg_p = _make_deg_kernel(Gd, Np)(dst_d, ew_d)
  d0 = deg_p[0][:, None]
  d1 = deg_p[1][:, None]

  RB = 1000
  grid = (N // RB,)

  y1 = pl.pallas_call(
      _tc_scale_body,
      grid=grid,
      in_specs=[_row_spec(RB, D), _row_spec(RB, 1), _row_spec(RB, 1)],
      out_specs=_row_spec(RB, D),
      out_shape=jax.ShapeDtypeStruct((Np, D), jnp.float32),
  )(x, d0, d1)

  prop = _make_prop_kernel(Gp, Np)
  S1 = prop(y1, pk, ew_p)

  y2 = pl.pallas_call(
      _tc_layer1_body,
      grid=grid,
      in_specs=[
          _row_spec(RB, D), _row_spec(RB, D),
          _row_spec(RB, 1), _row_spec(RB, 1),
          _full_spec((D, D)), _full_spec((1, D)),
      ],
      out_specs=_row_spec(RB, D),
      out_shape=jax.ShapeDtypeStruct((Np, D), jnp.float32),
  )(S1, y1, d0, d1, W1, b1[None, :])

  S2 = prop(y2, pk, ew_p)

  mu, lv = pl.pallas_call(
      _tc_layer23_body,
      grid=grid,
      in_specs=[
          _row_spec(RB, D), _row_spec(RB, D),
          _row_spec(RB, 1), _row_spec(RB, 1),
          _full_spec((D, Z)), _full_spec((1, Z)),
          _full_spec((D, Z)), _full_spec((1, Z)),
      ],
      out_specs=[_row_spec(RB, Z), _row_spec(RB, Z)],
      out_shape=[
          jax.ShapeDtypeStruct((N, Z), jnp.float32),
          jax.ShapeDtypeStruct((N, Z), jnp.float32),
      ],
  )(S2, y2, d0, d1, Wmu, bmu[None, :], Wlv, blv[None, :])

  return (mu, lv)
